# baseline (speedup 1.0000x reference)
import jax
import jax.numpy as jnp
from jax import lax
from jax.experimental import pallas as pl
from jax.experimental.pallas import tpu as pltpu
from jax.experimental.pallas import tpu_sc as plsc

_B = 32
_C = 64
_P = 128
_L = 16
_COLS = _P
_ROWS = _B * _C * _L
_BLOCK_ROWS = 8192

_N_WORKERS = 32
_ROWS_PER_W = _ROWS // _N_WORKERS
_CHUNK_ROWS = 32
_N_CHUNKS = _ROWS_PER_W // _CHUNK_ROWS


def _tc_mask_kernel(x_ref, moe_ref, meo_ref):
    x = x_ref[...]
    shape = x.shape
    row = jax.lax.broadcasted_iota(jnp.int32, shape, 0)
    col = jax.lax.broadcasted_iota(jnp.int32, shape, 1)
    oe = ((row // _L) ^ col) & 1
    zero = jnp.zeros_like(x)
    moe_ref[...] = jnp.where(oe == 1, zero, x)
    meo_ref[...] = jnp.where(oe == 0, zero, x)


def _sc_mask_writer(oe_hbm, eo_hbm, oe_buf, eo_buf, sem):
    lane = lax.broadcasted_iota(jnp.int32, (_L,), 0)
    v_odd = lane & 1
    v_even = v_odd ^ 1
    for r in range(_CHUNK_ROWS):
        top = r < 16
        a = v_odd if top else v_even
        b = v_even if top else v_odd
        for cseg in range(_COLS // _L):
            oe_buf[r, pl.ds(cseg * _L, _L)] = a
            eo_buf[r, pl.ds(cseg * _L, _L)] = b
    wid = lax.axis_index("s") * 2 + lax.axis_index("c")
    base = wid * _ROWS_PER_W
    copies = []
    for j in range(_N_CHUNKS):
        dst = pl.ds(base + j * _CHUNK_ROWS, _CHUNK_ROWS)
        copies.append(pltpu.async_copy(oe_buf, oe_hbm.at[dst, :], sem))
        copies.append(pltpu.async_copy(eo_buf, eo_hbm.at[dst, :], sem))
    for c in copies:
        c.wait()


def _sc_masks():
    mesh = plsc.VectorSubcoreMesh(core_axis_name="c", subcore_axis_name="s")
    f = pl.kernel(
        _sc_mask_writer,
        out_type=[
            jax.ShapeDtypeStruct((_ROWS, _COLS), jnp.int32),
            jax.ShapeDtypeStruct((_ROWS, _COLS), jnp.int32),
        ],
        mesh=mesh,
        scratch_types=[
            pltpu.VMEM((_CHUNK_ROWS, _COLS), jnp.int32),
            pltpu.VMEM((_CHUNK_ROWS, _COLS), jnp.int32),
            pltpu.SemaphoreType.DMA,
        ],
    )
    return f()


def kernel(inputs):
    x2d = jnp.transpose(inputs, (0, 1, 3, 2)).reshape(_ROWS, _COLS)
    oe2d, eo2d = _sc_masks()
    grid = (_ROWS // _BLOCK_ROWS,)
    spec = pl.BlockSpec((_BLOCK_ROWS, _COLS), lambda i: (i, 0))
    moe2d, meo2d = pl.pallas_call(
        _tc_mask_kernel,
        grid=grid,
        in_specs=[spec],
        out_specs=[spec, spec],
        out_shape=[
            jax.ShapeDtypeStruct((_ROWS, _COLS), jnp.float32),
            jax.ShapeDtypeStruct((_ROWS, _COLS), jnp.float32),
        ],
        compiler_params=pltpu.CompilerParams(
            dimension_semantics=("parallel",),
            skip_device_barrier=True,
        ),
    )(x2d)

    def _back(a):
        return jnp.transpose(a.reshape(_B, _C, _L, _P), (0, 1, 3, 2))

    return _back(moe2d), _back(meo2d), _back(oe2d), _back(eo2d)


# speedup vs baseline: 1.5762x; 1.5762x over previous
import jax
import jax.numpy as jnp
from jax.experimental import pallas as pl
from jax.experimental.pallas import tpu as pltpu

_B = 32
_C = 64
_P = 128
_L = 16
_COLS = _P
_ROWS = _B * _C * _L
_IN_BLOCK = 8192
_OUT_BLOCK = 4096


def _mask_kernel(x_ref, moe_ref, meo_ref, oe_ref, eo_ref):
    i = pl.program_id(0)
    sub = i % (_IN_BLOCK // _OUT_BLOCK)
    x = x_ref[pl.ds(sub * _OUT_BLOCK, _OUT_BLOCK), :]
    shape = x.shape
    row = jax.lax.broadcasted_iota(jnp.int32, shape, 0)
    col = jax.lax.broadcasted_iota(jnp.int32, shape, 1)
    oe = ((row // _L) ^ col) & 1
    eo = oe ^ 1
    oe_ref[...] = oe
    eo_ref[...] = eo
    zero = jnp.zeros_like(x)
    moe_ref[...] = jnp.where(oe == 1, zero, x)
    meo_ref[...] = jnp.where(oe == 0, zero, x)


def kernel(inputs):
    x2d = jnp.transpose(inputs, (0, 1, 3, 2)).reshape(_ROWS, _COLS)
    grid = (_ROWS // _OUT_BLOCK,)
    ratio = _IN_BLOCK // _OUT_BLOCK
    in_spec = pl.BlockSpec((_IN_BLOCK, _COLS), lambda i: (i // ratio, 0))
    out_spec = pl.BlockSpec((_OUT_BLOCK, _COLS), lambda i: (i, 0))
    out = pl.pallas_call(
        _mask_kernel,
        grid=grid,
        in_specs=[in_spec],
        out_specs=[out_spec, out_spec, out_spec, out_spec],
        out_shape=[
            jax.ShapeDtypeStruct((_ROWS, _COLS), jnp.float32),
            jax.ShapeDtypeStruct((_ROWS, _COLS), jnp.float32),
            jax.ShapeDtypeStruct((_ROWS, _COLS), jnp.int32),
            jax.ShapeDtypeStruct((_ROWS, _COLS), jnp.int32),
        ],
    )(x2d)

    def _back(a):
        return jnp.transpose(a.reshape(_B, _C, _L, _P), (0, 1, 3, 2))

    return tuple(_back(a) for a in out)


# speedup vs baseline: 1.6110x; 1.0221x over previous
import jax
import jax.numpy as jnp
from jax.experimental import pallas as pl
from jax.experimental.pallas import tpu as pltpu

_B = 32
_C = 64
_P = 128
_L = 16
_COLS = _P
_ROWS = _B * _C * _L
_BLOCK_ROWS = 8192


def _mask_kernel(x_ref, moe_ref, meo_ref, oe_ref, eo_ref):
    x = x_ref[...]
    shape = x.shape
    row = jax.lax.broadcasted_iota(jnp.int32, shape, 0)
    col = jax.lax.broadcasted_iota(jnp.int32, shape, 1)
    oe = ((row // _L) ^ col) & 1
    eo = oe ^ 1
    oe_ref[...] = oe
    eo_ref[...] = eo
    zero = jnp.zeros_like(x)
    moe_ref[...] = jnp.where(oe == 1, zero, x)
    meo_ref[...] = jnp.where(oe == 0, zero, x)


def kernel(inputs):
    x2d = jnp.transpose(inputs, (0, 1, 3, 2)).reshape(_ROWS, _COLS)
    grid = (_ROWS // _BLOCK_ROWS,)
    spec = pl.BlockSpec((_BLOCK_ROWS, _COLS), lambda i: (i, 0))
    out = pl.pallas_call(
        _mask_kernel,
        grid=grid,
        in_specs=[spec],
        out_specs=[spec, spec, spec, spec],
        out_shape=[
            jax.ShapeDtypeStruct((_ROWS, _COLS), jnp.float32),
            jax.ShapeDtypeStruct((_ROWS, _COLS), jnp.float32),
            jax.ShapeDtypeStruct((_ROWS, _COLS), jnp.int32),
            jax.ShapeDtypeStruct((_ROWS, _COLS), jnp.int32),
        ],
        compiler_params=pltpu.CompilerParams(
            dimension_semantics=("parallel",),
        ),
    )(x2d)

    def _back(a):
        return jnp.transpose(a.reshape(_B, _C, _L, _P), (0, 1, 3, 2))

    return tuple(_back(a) for a in out)
